# Initial kernel scaffold; baseline (speedup 1.0000x reference)
import jax
import jax.numpy as jnp
from jax.experimental import pallas as pl


def kernel(table, indices):
    raise NotImplementedError("write your pallas kernel here")


# speedup vs baseline: 2.9778x; 2.9778x over previous
import functools

import jax
import jax.numpy as jnp
from jax import lax
from jax.experimental import pallas as pl
from jax.experimental.pallas import tpu as pltpu
from jax.experimental.pallas import tpu_sc as plsc

D = 128
TOTAL = 4096 * 50
NC, NS = 2, 16
NW = NC * NS
PER_W = TOTAL // NW
CHUNK = 128
NCHUNK = PER_W // CHUNK


def _sc_gather(table, idx3):
    mesh = plsc.VectorSubcoreMesh(core_axis_name="c", subcore_axis_name="s")

    @functools.partial(
        pl.kernel,
        out_type=jax.ShapeDtypeStruct((TOTAL, D), jnp.float32),
        mesh=mesh,
        scratch_types=[
            pltpu.VMEM((NCHUNK, CHUNK), jnp.int32),
            pltpu.VMEM((CHUNK, D), jnp.float32),
            pltpu.SemaphoreType.DMA,
        ],
    )
    def k(table_hbm, idx_hbm, out_hbm, idx_v, rows_v, sem):
        wid = lax.axis_index("s") * NC + lax.axis_index("c")
        pltpu.sync_copy(idx_hbm.at[wid], idx_v)
        base = wid * PER_W

        def body(c, carry):
            pltpu.async_copy(table_hbm.at[idx_v.at[c]], rows_v, sem).wait()
            pltpu.sync_copy(rows_v, out_hbm.at[pl.ds(base + c * CHUNK, CHUNK)])
            return carry

        lax.fori_loop(0, NCHUNK, body, 0)

    return k(table, idx3)


def kernel(table, indices):
    idx3 = indices.reshape(-1).astype(jnp.int32).reshape(NW, NCHUNK, CHUNK)
    out = _sc_gather(table, idx3)
    return out.reshape(indices.shape[0], indices.shape[1], D)


# speedup vs baseline: 3.3545x; 1.1265x over previous
import functools

import jax
import jax.numpy as jnp
from jax import lax
from jax.experimental import pallas as pl
from jax.experimental.pallas import tpu as pltpu
from jax.experimental.pallas import tpu_sc as plsc

D = 128
TOTAL = 4096 * 50
NC, NS = 2, 16
NW = NC * NS
PER_W = TOTAL // NW
CHUNK = 128
NCHUNK = PER_W // CHUNK
K = 2
NG = NCHUNK // K
NBUF = 2 * K


def _sc_gather(table, idx3):
    mesh = plsc.VectorSubcoreMesh(core_axis_name="c", subcore_axis_name="s")

    @functools.partial(
        pl.kernel,
        out_type=jax.ShapeDtypeStruct((TOTAL, D), jnp.float32),
        mesh=mesh,
        scratch_types=[
            pltpu.VMEM((NCHUNK, CHUNK), jnp.int32),
            pltpu.VMEM((NBUF, CHUNK, D), jnp.float32),
            pltpu.SemaphoreType.DMA,
            pltpu.SemaphoreType.DMA,
        ],
    )
    def k(table_hbm, idx_hbm, out_hbm, idx_v, rows_v, gsem, osem):
        wid = lax.axis_index("s") * NC + lax.axis_index("c")
        pltpu.sync_copy(idx_hbm.at[wid], idx_v)
        base = wid * PER_W

        def gather(c, b):
            return pltpu.make_async_copy(
                table_hbm.at[idx_v.at[c]], rows_v.at[b], gsem)

        def writeback(c, b):
            return pltpu.make_async_copy(
                rows_v.at[b], out_hbm.at[pl.ds(base + c * CHUNK, CHUNK)], osem)

        for j in range(K):
            gather(j, j).start()

        def body(g, carry):
            s = (g % 2) * K
            t = ((g + 1) % 2) * K

            @pl.when(g >= 1)
            def _():
                for j in range(K):
                    writeback((g - 1) * K + j, t + j).wait()

            @pl.when(g + 1 < NG)
            def _():
                for j in range(K):
                    gather((g + 1) * K + j, t + j).start()

            for j in range(K):
                gather(g * K + j, s + j).wait()
            for j in range(K):
                writeback(g * K + j, s + j).start()
            return carry

        lax.fori_loop(0, NG, body, 0)

        s = ((NG - 1) % 2) * K
        for j in range(K):
            writeback((NG - 1) * K + j, s + j).wait()

    return k(table, idx3)


def kernel(table, indices):
    idx3 = indices.reshape(-1).astype(jnp.int32).reshape(NW, NCHUNK, CHUNK)
    out = _sc_gather(table, idx3)
    return out.reshape(indices.shape[0], indices.shape[1], D)


# speedup vs baseline: 10.7525x; 3.2054x over previous
import functools

import jax
import jax.numpy as jnp
from jax import lax
from jax.experimental import pallas as pl
from jax.experimental.pallas import tpu as pltpu
from jax.experimental.pallas import tpu_sc as plsc

VOCAB = 100000
D = 128
B = 4096
H = 50
NC, NS = 2, 16
NW = NC * NS
BW = B // NW
K = 2
NG = H // K
NBUF = 2 * K


def _sc_gather(table, idx_t):
    mesh = plsc.VectorSubcoreMesh(core_axis_name="c", subcore_axis_name="s")

    @functools.partial(
        pl.kernel,
        out_type=jax.ShapeDtypeStruct((H, B, D), jnp.float32),
        mesh=mesh,
        scratch_types=[
            pltpu.VMEM((H, BW), jnp.int32),
            pltpu.VMEM((NBUF, BW, D), jnp.float32),
            pltpu.SemaphoreType.DMA,
            pltpu.SemaphoreType.DMA,
        ],
    )
    def k(table_hbm, idx_hbm, out_hbm, idx_v, rows_v, gsem, osem):
        wid = lax.axis_index("s") * NC + lax.axis_index("c")
        row0 = wid * BW

        pltpu.sync_copy(idx_hbm.at[:, wid], idx_v)

        def gather(h, s):
            return pltpu.make_async_copy(
                table_hbm.at[idx_v.at[h]], rows_v.at[s], gsem)

        def writeback(h, s):
            return pltpu.make_async_copy(
                rows_v.at[s], out_hbm.at[h, pl.ds(row0, BW)], osem)

        for j in range(K):
            gather(j, j).start()

        def body(g, carry):
            s = (g % 2) * K
            t = ((g + 1) % 2) * K

            @pl.when(g >= 1)
            def _():
                for j in range(K):
                    writeback((g - 1) * K + j, t + j).wait()

            @pl.when(g + 1 < NG)
            def _():
                for j in range(K):
                    gather((g + 1) * K + j, t + j).start()

            for j in range(K):
                gather(g * K + j, s + j).wait()
            for j in range(K):
                writeback(g * K + j, s + j).start()
            return carry

        lax.fori_loop(0, NG, body, 0)

        s = ((NG - 1) % 2) * K
        for j in range(K):
            writeback((NG - 1) * K + j, s + j).wait()

    return k(table, idx_t)


def kernel(table, indices):
    idx_t = indices.astype(jnp.int32).T.reshape(H, NW, BW)
    idx_t = jnp.where((idx_t >= 0) & (idx_t < VOCAB), idx_t, VOCAB)
    out = _sc_gather(table, idx_t)
    return out.transpose(1, 0, 2)


# speedup vs baseline: 10.8106x; 1.0054x over previous
import functools

import jax
import jax.numpy as jnp
from jax import lax
from jax.experimental import pallas as pl
from jax.experimental.pallas import tpu as pltpu
from jax.experimental.pallas import tpu_sc as plsc

VOCAB = 100000
D = 128
B = 4096
H = 50
NC, NS = 2, 16
NW = NC * NS
BW = B // NW
K = 2
NG = H // K
NBUF = 3 * K


def _sc_gather(table, idx_t):
    mesh = plsc.VectorSubcoreMesh(core_axis_name="c", subcore_axis_name="s")

    @functools.partial(
        pl.kernel,
        out_type=jax.ShapeDtypeStruct((H, B, D), jnp.float32),
        mesh=mesh,
        scratch_types=[
            pltpu.VMEM((H, BW), jnp.int32),
            pltpu.VMEM((NBUF, BW, D), jnp.float32),
            pltpu.SemaphoreType.DMA,
            pltpu.SemaphoreType.DMA,
        ],
    )
    def k(table_hbm, idx_hbm, out_hbm, idx_v, rows_v, gsem, osem):
        wid = lax.axis_index("s") * NC + lax.axis_index("c")
        row0 = wid * BW

        pltpu.sync_copy(idx_hbm.at[:, wid], idx_v)

        def gather(h, s):
            return pltpu.make_async_copy(
                table_hbm.at[idx_v.at[h]], rows_v.at[s], gsem)

        def writeback(h, s):
            return pltpu.make_async_copy(
                rows_v.at[s], out_hbm.at[h, pl.ds(row0, BW)], osem)

        for j in range(K):
            gather(j, j).start()

        def body(g, carry):
            s = (g % 3) * K
            t = ((g + 1) % 3) * K

            @pl.when(g >= 2)
            def _():
                for j in range(K):
                    writeback((g - 2) * K + j, t + j).wait()

            @pl.when(g + 1 < NG)
            def _():
                for j in range(K):
                    gather((g + 1) * K + j, t + j).start()

            for j in range(K):
                gather(g * K + j, s + j).wait()
            for j in range(K):
                writeback(g * K + j, s + j).start()
            return carry

        lax.fori_loop(0, NG, body, 0)

        for gg in (NG - 2, NG - 1):
            for j in range(K):
                writeback(gg * K + j, (gg % 3) * K + j).wait()

    return k(table, idx_t)


def kernel(table, indices):
    idx_t = indices.astype(jnp.int32).T.reshape(H, NW, BW)
    idx_t = jnp.where((idx_t >= 0) & (idx_t < VOCAB), idx_t, VOCAB)
    out = _sc_gather(table, idx_t)
    return out.transpose(1, 0, 2)
